# baseline (speedup 1.0000x reference)
import functools

import jax
import jax.numpy as jnp
from jax import lax
from jax.experimental import pallas as pl
from jax.experimental.pallas import tpu as pltpu
from jax.experimental.pallas import tpu_sc as plsc

B = 4
N = 10000
E = 320000
DV = 128
DE = 16
DC = 32

GT = 2
CT = 2500

SC_CT = 1200
QW = 4
TQ = SC_CT // QW
ECH_T = 25
NCH = TQ // ECH_T

TC_CT = CT - SC_CT
ECT = 650
NEB = TC_CT // ECT
VCH = 5000
NVB = N // VCH
NVSTEP = B * NVB
NESTEP = B * GT * NEB
NSTEPS = NVSTEP + NESTEP

_sc_mesh = plsc.VectorSubcoreMesh(core_axis_name="c", subcore_axis_name="s")


@functools.partial(
    pl.kernel,
    out_type=jax.ShapeDtypeStruct((B, GT, QW, 8, 16), jnp.float32),
    mesh=_sc_mesh,
    scratch_types=[
        pltpu.VMEM((2, ECH_T, 8, 128), jnp.float32),
        pltpu.VMEM((8, 16), jnp.float32),
        pltpu.SemaphoreType.DMA,
        pltpu.SemaphoreType.DMA,
    ],
    compiler_params=pltpu.CompilerParams(use_tc_tiling_on_sc=False),
)
def _sc_edge(e_hbm, eout_hbm, ebuf, est, sem0, sem1):
    wid = lax.axis_index("s") * 2 + lax.axis_index("c")
    bi = wid // (GT * QW)
    gi = (wid // QW) % GT
    qi = wid % QW
    sems = (sem0, sem1)
    tbase = TC_CT + qi * TQ

    for c in range(2):
        pltpu.async_copy(
            e_hbm.at[bi, gi, pl.ds(tbase + c * ECH_T, ECH_T), :, :],
            ebuf.at[c], sems[c])

    def _chunk(slot, accs):
        def body(t, a):
            return tuple(
                a[s * 8 + m] + ebuf[slot, t, s, pl.ds(m * 16, 16)]
                for s in range(8) for m in range(8))
        return lax.fori_loop(0, ECH_T, body, accs)

    accs = tuple(jnp.zeros((16,), jnp.float32) for _ in range(64))
    for c in range(NCH):
        slot = c % 2
        pltpu.make_async_copy(
            e_hbm.at[bi, gi, pl.ds(tbase + c * ECH_T, ECH_T), :, :],
            ebuf.at[slot], sems[slot]).wait()
        accs = _chunk(slot, accs)
        if c + 2 < NCH:
            pltpu.async_copy(
                e_hbm.at[bi, gi, pl.ds(tbase + (c + 2) * ECH_T, ECH_T), :, :],
                ebuf.at[slot], sems[slot])

    for s in range(8):
        tot = accs[s * 8]
        for m in range(1, 8):
            tot = tot + accs[s * 8 + m]
        est[s, :] = tot
    pltpu.sync_copy(est, eout_hbm.at[bi, gi, qi])


def _tc_reduce(v_ref, e_ref, vsum_ref, esum_ref, acc_v, acc_e):
    i = pl.program_id(0)

    @pl.when(i == 0)
    def _init():
        acc_v[...] = jnp.zeros_like(acc_v)
        acc_e[...] = jnp.zeros_like(acc_e)

    @pl.when(i < NVSTEP)
    def _vstep():
        acc_v[i // NVB] += jnp.sum(v_ref[0].reshape(VCH // 8, 8, 128), axis=0)

    @pl.when(i >= NVSTEP)
    def _estep():
        j = i - NVSTEP
        acc_e[j // (GT * NEB), (j % (GT * NEB)) // NEB] += (
            jnp.sum(e_ref[0, 0], axis=0))

    @pl.when(i == NSTEPS - 1)
    def _out():
        vsum_ref[...] = acc_v[...]
        esum_ref[...] = acc_e[...]


def _tc_finish(ctx_ref, vs_ref, es_ref, scp_ref, w_ref, b_ref, out_ref):
    v_agg = jnp.sum(vs_ref[...], axis=1) * (1.0 / N)
    e_tc = jnp.sum(es_ref[...], axis=3)
    e_sc = jnp.sum(scp_ref[...], axis=(2, 4))
    e_agg = (e_tc + e_sc).reshape(B, DE) * (1.0 / E)
    ctx = ctx_ref[...][:, 0, :]
    w = w_ref[...]
    out = (
        jnp.dot(ctx, w[:DC], preferred_element_type=jnp.float32)
        + jnp.dot(v_agg, w[DC:DC + DV], preferred_element_type=jnp.float32)
        + jnp.dot(e_agg, w[DC + DV:], preferred_element_type=jnp.float32)
        + b_ref[...][None, :]
    )
    out_ref[...] = out[:, None, :]


@jax.jit
def kernel(context, vertex_data, edge_data, W, b):
    e_view = edge_data.reshape(B, CT, 128, GT, 8).transpose(0, 3, 1, 4, 2)

    sc_part = _sc_edge(e_view)

    def emap(i):
        j = jnp.maximum(i - NVSTEP, 0)
        return (j // (GT * NEB), (j % (GT * NEB)) // NEB, j % NEB, 0, 0)

    def vmap_(i):
        k = jnp.minimum(i, NVSTEP - 1)
        return (k // NVB, k % NVB, 0)

    vsum, esum = pl.pallas_call(
        _tc_reduce,
        grid=(NSTEPS,),
        in_specs=[
            pl.BlockSpec((1, VCH, DV), vmap_),
            pl.BlockSpec((1, 1, ECT, 8, 128), emap),
        ],
        out_specs=[
            pl.BlockSpec((B, 8, DV), lambda i: (0, 0, 0)),
            pl.BlockSpec((B, GT, 8, 128), lambda i: (0, 0, 0, 0)),
        ],
        out_shape=[
            jax.ShapeDtypeStruct((B, 8, DV), jnp.float32),
            jax.ShapeDtypeStruct((B, GT, 8, 128), jnp.float32),
        ],
        scratch_shapes=[
            pltpu.VMEM((B, 8, DV), jnp.float32),
            pltpu.VMEM((B, GT, 8, 128), jnp.float32),
        ],
        compiler_params=pltpu.CompilerParams(
            dimension_semantics=("arbitrary",),
        ),
    )(vertex_data, e_view)

    return pl.pallas_call(
        _tc_finish,
        out_shape=jax.ShapeDtypeStruct((B, 1, DC), jnp.float32),
    )(context, vsum, esum, sc_part, W, b)


# speedup vs baseline: 1.3305x; 1.3305x over previous
import jax
import jax.numpy as jnp
from jax.experimental import pallas as pl
from jax.experimental.pallas import tpu as pltpu

B = 4
N = 10000
E = 320000
DV = 128
DE = 16
DC = 32

GT = 2
CT = 2500

VCH = 10000
NVB = N // VCH
ECT = 2500
NEB = CT // ECT
NVSTEP = B * NVB
NESTEP = B * GT * NEB
NSTEPS = NVSTEP + NESTEP


def _tc_kernel(ctx_ref, v_ref, e_ref, w_ref, b_ref, out_ref, acc_v, acc_e):
    i = pl.program_id(0)

    @pl.when(i == 0)
    def _init():
        acc_v[...] = jnp.zeros_like(acc_v)
        acc_e[...] = jnp.zeros_like(acc_e)

    @pl.when(i < NVSTEP)
    def _vstep():
        bi = i // NVB
        acc_v[bi] += jnp.sum(v_ref[0].reshape(VCH // 8, 8, 128), axis=0)

    @pl.when(i >= NVSTEP)
    def _estep():
        j = i - NVSTEP
        bi = j // (GT * NEB)
        gi = (j % (GT * NEB)) // NEB
        acc_e[bi, gi] += jnp.sum(e_ref[0, 0], axis=0)

    @pl.when(i == NSTEPS - 1)
    def _final():
        v_agg = jnp.sum(acc_v[...], axis=1) * (1.0 / N)
        e_agg = jnp.sum(acc_e[...], axis=3).reshape(B, DE) * (1.0 / E)
        ctx = ctx_ref[...][:, 0, :]
        w = w_ref[...]
        out = (
            jnp.dot(ctx, w[:DC], preferred_element_type=jnp.float32)
            + jnp.dot(v_agg, w[DC:DC + DV], preferred_element_type=jnp.float32)
            + jnp.dot(e_agg, w[DC + DV:], preferred_element_type=jnp.float32)
            + b_ref[...][None, :]
        )
        out_ref[...] = out[:, None, :]


@jax.jit
def kernel(context, vertex_data, edge_data, W, b):
    e_view = edge_data.reshape(B, CT, 128, GT, 8).transpose(0, 3, 1, 4, 2)
    grid = (NSTEPS,)

    def emap(i):
        j = jnp.maximum(i - NVSTEP, 0)
        return (j // (GT * NEB), (j % (GT * NEB)) // NEB, j % NEB, 0, 0)

    def vmap_(i):
        k = jnp.minimum(i, NVSTEP - 1)
        return (k // NVB, k % NVB, 0)

    return pl.pallas_call(
        _tc_kernel,
        grid=grid,
        in_specs=[
            pl.BlockSpec((B, 1, DC), lambda i: (0, 0, 0)),
            pl.BlockSpec((1, VCH, DV), vmap_),
            pl.BlockSpec((1, 1, ECT, 8, 128), emap),
            pl.BlockSpec((DC + DV + DE, DC), lambda i: (0, 0)),
            pl.BlockSpec((DC,), lambda i: (0,)),
        ],
        out_specs=pl.BlockSpec((B, 1, DC), lambda i: (0, 0, 0)),
        out_shape=jax.ShapeDtypeStruct((B, 1, DC), jnp.float32),
        scratch_shapes=[
            pltpu.VMEM((B, 8, DV), jnp.float32),
            pltpu.VMEM((B, GT, 8, 128), jnp.float32),
        ],
        compiler_params=pltpu.CompilerParams(
            dimension_semantics=("arbitrary",),
            vmem_limit_bytes=34 * 1024 * 1024,
        ),
    )(context, vertex_data, e_view, W, b)


# speedup vs baseline: 1.5384x; 1.1563x over previous
import jax
import jax.numpy as jnp
from jax.experimental import pallas as pl
from jax.experimental.pallas import tpu as pltpu

B = 4
N = 10000
E = 320000
DV = 128
DE = 16
DC = 32

GT = 2
CT = 2500

VCH = 10000
NVB = N // VCH
ECT = 2500
NEB = CT // ECT
NVSTEP = B * NVB
NESTEP = B * NEB
NSTEPS = NVSTEP + NESTEP


def _tc_kernel(ctx_ref, v_ref, e_ref, w_ref, b_ref, out_ref, acc_v, acc_e):
    i = pl.program_id(0)

    @pl.when(i == 0)
    def _init():
        acc_v[...] = jnp.zeros_like(acc_v)
        acc_e[...] = jnp.zeros_like(acc_e)

    @pl.when(i < NVSTEP)
    def _vstep():
        bi = i // NVB
        acc_v[bi] += jnp.sum(v_ref[0].reshape(VCH // 8, 8, 128), axis=0)

    @pl.when(i >= NVSTEP)
    def _estep():
        j = i - NVSTEP
        acc_e[j] += jnp.sum(e_ref[0], axis=1)

    @pl.when(i == NSTEPS - 1)
    def _final():
        v_agg = jnp.sum(acc_v[...], axis=1) * (1.0 / N)
        e_agg = jnp.sum(acc_e[...], axis=3).reshape(B, DE) * (1.0 / E)
        ctx = ctx_ref[...][:, 0, :]
        w = w_ref[...]
        out = (
            jnp.dot(ctx, w[:DC], preferred_element_type=jnp.float32)
            + jnp.dot(v_agg, w[DC:DC + DV], preferred_element_type=jnp.float32)
            + jnp.dot(e_agg, w[DC + DV:], preferred_element_type=jnp.float32)
            + b_ref[...][None, :]
        )
        out_ref[...] = out[:, None, :]


@jax.jit
def kernel(context, vertex_data, edge_data, W, b):
    e_view = edge_data.reshape(B, CT, 128, GT, 8).transpose(0, 3, 1, 4, 2)
    grid = (NSTEPS,)

    def emap(i):
        j = jnp.maximum(i - NVSTEP, 0)
        return (j, 0, 0, 0, 0)

    def vmap_(i):
        k = jnp.minimum(i, NVSTEP - 1)
        return (k // NVB, k % NVB, 0)

    return pl.pallas_call(
        _tc_kernel,
        grid=grid,
        in_specs=[
            pl.BlockSpec((B, 1, DC), lambda i: (0, 0, 0)),
            pl.BlockSpec((1, VCH, DV), vmap_),
            pl.BlockSpec((1, GT, ECT, 8, 128), emap),
            pl.BlockSpec((DC + DV + DE, DC), lambda i: (0, 0)),
            pl.BlockSpec((DC,), lambda i: (0,)),
        ],
        out_specs=pl.BlockSpec((B, 1, DC), lambda i: (0, 0, 0)),
        out_shape=jax.ShapeDtypeStruct((B, 1, DC), jnp.float32),
        scratch_shapes=[
            pltpu.VMEM((B, 8, DV), jnp.float32),
            pltpu.VMEM((B, GT, 8, 128), jnp.float32),
        ],
        compiler_params=pltpu.CompilerParams(
            dimension_semantics=("arbitrary",),
            vmem_limit_bytes=56 * 1024 * 1024,
        ),
    )(context, vertex_data, e_view, W, b)
